# baseline (speedup 1.0000x reference)
import functools

import jax
import jax.numpy as jnp
from jax import lax
from jax.experimental import pallas as pl
from jax.experimental.pallas import tpu as pltpu
from jax.experimental.pallas import tpu_sc as plsc

TM = 256


def _gelu_exact(x):
    return 0.5 * x * (1.0 + lax.erf(x * 0.7071067811865475))


def _router_body(x_ref, g_ref, b_ref, wr_ref, br_ref,
                 h_ref, gates_ref, idx_ref, *, n_experts):
    xv = x_ref[...]
    mu = jnp.mean(xv, axis=-1, keepdims=True)
    xc = xv - mu
    var = jnp.mean(xc * xc, axis=-1, keepdims=True)
    h = xc * lax.rsqrt(var + 1e-5) * g_ref[0, :] + b_ref[0, :]
    h_ref[...] = h.astype(jnp.bfloat16)
    logits = jnp.dot(h, wr_ref[...],
                     preferred_element_type=jnp.float32) + br_ref[0, :]
    iota = lax.broadcasted_iota(jnp.int32, logits.shape, 1)
    v1 = jnp.max(logits, axis=-1, keepdims=True)
    i1 = jnp.min(jnp.where(logits >= v1, iota, n_experts),
                 axis=-1, keepdims=True)
    l2 = jnp.where(iota == i1, jnp.float32(-1e30), logits)
    v2 = jnp.max(l2, axis=-1, keepdims=True)
    i2 = jnp.min(jnp.where(l2 >= v2, iota, n_experts),
                 axis=-1, keepdims=True)
    g1 = 1.0 / (1.0 + jnp.exp(v2 - v1))
    col = lax.broadcasted_iota(jnp.int32, (xv.shape[0], 2), 1)
    gates_ref[...] = jnp.where(col == 0, g1, 1.0 - g1)
    idx_ref[...] = jnp.where(col == 0, i1, i2)


def _router(x2, ln_g, ln_b, Wr, br):
    N, D = x2.shape
    E = Wr.shape[1]
    return pl.pallas_call(
        functools.partial(_router_body, n_experts=E),
        out_shape=[
            jax.ShapeDtypeStruct((N, D), jnp.bfloat16),
            jax.ShapeDtypeStruct((N, 2), jnp.float32),
            jax.ShapeDtypeStruct((N, 2), jnp.int32),
        ],
    )(x2, ln_g.reshape(1, D), ln_b.reshape(1, D), Wr, br.reshape(1, E))


def _sc_gather_rows(hw, tok, P):
    N, DW = hw.shape
    info = plsc.get_sparse_core_info()
    NC, NS = info.num_cores, info.num_subcores
    NW = NC * NS
    b_per_w = P // NW
    CH = 80
    n_chunks = b_per_w // CH

    def body(h_hbm, tok_hbm, out_hbm, idx_v, buf0, buf1, sem0, sem1):
        wid = lax.axis_index("s") * NC + lax.axis_index("c")
        base = wid * b_per_w
        pltpu.sync_copy(tok_hbm.at[pl.ds(base, b_per_w)], idx_v)
        bufs, sems = (buf0, buf1), (sem0, sem1)
        pend = [None, None]

        def start(i):
            b = i % 2
            pend[b] = pltpu.async_copy(
                h_hbm.at[idx_v.at[pl.ds(i * CH, CH)]], bufs[b], sems[b])

        start(0)
        for i in range(n_chunks):
            if i + 1 < n_chunks:
                start(i + 1)
            b = i % 2
            pend[b].wait()
            pltpu.sync_copy(bufs[b], out_hbm.at[pl.ds(base + i * CH, CH)])

    return pl.kernel(
        body,
        out_type=jax.ShapeDtypeStruct((P, DW), jnp.float32),
        mesh=plsc.VectorSubcoreMesh(core_axis_name="c", subcore_axis_name="s"),
        scratch_types=[
            pltpu.VMEM((b_per_w,), jnp.int32),
            pltpu.VMEM((CH, DW), jnp.float32),
            pltpu.VMEM((CH, DW), jnp.float32),
            pltpu.SemaphoreType.DMA,
            pltpu.SemaphoreType.DMA,
        ],
    )(hw, tok)


def _ffn_body(te_ref, h_ref, w1_ref, b1_ref, w2_ref, b2_ref, g_ref, z_ref):
    hid = jnp.dot(h_ref[...], w1_ref[0], preferred_element_type=jnp.float32)
    hid = _gelu_exact(hid + b1_ref[0, 0, :])
    y = jnp.dot(hid.astype(jnp.bfloat16), w2_ref[0],
                preferred_element_type=jnp.float32)
    z_ref[...] = (y + b2_ref[0, 0, :]) * g_ref[...]


def _grouped_ffn(h_sorted, tile_expert, g_sorted, W1b, b1, W2b, b2):
    P, D = h_sorted.shape
    E, _, DFF = W1b.shape
    NT = P // TM
    grid_spec = pltpu.PrefetchScalarGridSpec(
        num_scalar_prefetch=1,
        grid=(NT,),
        in_specs=[
            pl.BlockSpec((TM, D), lambda m, te: (m, 0)),
            pl.BlockSpec((1, D, DFF), lambda m, te: (te[m], 0, 0)),
            pl.BlockSpec((1, 1, DFF), lambda m, te: (te[m], 0, 0)),
            pl.BlockSpec((1, DFF, D), lambda m, te: (te[m], 0, 0)),
            pl.BlockSpec((1, 1, D), lambda m, te: (te[m], 0, 0)),
            pl.BlockSpec((TM, 1), lambda m, te: (m, 0)),
        ],
        out_specs=pl.BlockSpec((TM, D), lambda m, te: (m, 0)),
    )
    return pl.pallas_call(
        _ffn_body,
        grid_spec=grid_spec,
        out_shape=jax.ShapeDtypeStruct((P, D), jnp.float32),
        compiler_params=pltpu.CompilerParams(
            dimension_semantics=("arbitrary",),
            vmem_limit_bytes=100 * 1024 * 1024,
        ),
    )(tile_expert, h_sorted, W1b, b1.reshape(E, 1, DFF), W2b,
      b2.reshape(E, 1, D), g_sorted)


def _sc_combine(x2, z, pos_flat):
    N, D = x2.shape
    info = plsc.get_sparse_core_info()
    NC, NS, L = info.num_cores, info.num_subcores, info.num_lanes
    NW = NC * NS
    b_per_w = N // NW
    CH = 16
    n_chunks = b_per_w // CH
    n_lane_blocks = D // L

    def body(x_hbm, z_hbm, pos_hbm, out_hbm,
             ip_v, xb0, xb1, zb0, zb1, sx0, sx1, sz0, sz1):
        wid = lax.axis_index("s") * NC + lax.axis_index("c")
        base = wid * b_per_w
        pltpu.sync_copy(pos_hbm.at[pl.ds(2 * base, 2 * b_per_w)], ip_v)
        xbufs, zbufs = (xb0, xb1), (zb0, zb1)
        xsems, zsems = (sx0, sx1), (sz0, sz1)
        xpend = [None, None]
        zpend = [None, None]

        def start(i):
            b = i % 2
            xpend[b] = pltpu.async_copy(
                x_hbm.at[pl.ds(base + i * CH, CH)], xbufs[b], xsems[b])
            zpend[b] = pltpu.async_copy(
                z_hbm.at[ip_v.at[pl.ds(2 * i * CH, 2 * CH)]],
                zbufs[b], zsems[b])

        start(0)
        for i in range(n_chunks):
            if i + 1 < n_chunks:
                start(i + 1)
            b = i % 2
            xpend[b].wait()
            zpend[b].wait()
            xb, zb = xbufs[b], zbufs[b]

            def row_add(r, carry):
                for c in range(n_lane_blocks):
                    s = pl.ds(c * L, L)
                    xb[r, s] = xb[r, s] + zb[2 * r, s] + zb[2 * r + 1, s]
                return carry

            lax.fori_loop(0, CH, row_add, 0)
            pltpu.sync_copy(xb, out_hbm.at[pl.ds(base + i * CH, CH)])

    return pl.kernel(
        body,
        out_type=jax.ShapeDtypeStruct((N, D), jnp.float32),
        mesh=plsc.VectorSubcoreMesh(core_axis_name="c", subcore_axis_name="s"),
        scratch_types=[
            pltpu.VMEM((2 * b_per_w,), jnp.int32),
            pltpu.VMEM((CH, D), jnp.float32),
            pltpu.VMEM((CH, D), jnp.float32),
            pltpu.VMEM((2 * CH, D), jnp.float32),
            pltpu.VMEM((2 * CH, D), jnp.float32),
            pltpu.SemaphoreType.DMA,
            pltpu.SemaphoreType.DMA,
            pltpu.SemaphoreType.DMA,
            pltpu.SemaphoreType.DMA,
        ],
    )(x2, z, pos_flat)


def kernel(x, ln_g, ln_b, Wr, br, W1, b1, W2, b2):
    B, T, D = x.shape
    E = Wr.shape[1]
    DFF = W1.shape[2]
    N = B * T
    K = 2
    NK = N * K
    P = NK + E * TM

    x2 = x.reshape(N, D)
    h, gates, idx = _router(x2, ln_g, ln_b, Wr, br)

    idx_flat = idx.reshape(NK)
    onehot = (idx_flat[:, None] == jnp.arange(E, dtype=jnp.int32)
              ).astype(jnp.int32)
    counts = jnp.sum(onehot, axis=0)
    ranks = jnp.sum(onehot * (jnp.cumsum(onehot, axis=0) - onehot),
                    axis=1)
    padded = ((counts + TM - 1) // TM) * TM
    off = jnp.concatenate([jnp.zeros((1,), jnp.int32),
                           jnp.cumsum(padded)[:-1].astype(jnp.int32)])
    pos = off[idx_flat] + ranks
    row_tok = jnp.zeros((P,), jnp.int32).at[pos].set(
        jnp.arange(NK, dtype=jnp.int32) // K)
    g_sorted = jnp.zeros((P,), jnp.float32).at[pos].set(
        gates.reshape(NK)).reshape(P, 1)
    bnd = jnp.cumsum(padded // TM)
    t_iota = jnp.arange(P // TM, dtype=jnp.int32)
    tile_expert = jnp.minimum(
        jnp.sum((t_iota[:, None] >= bnd[None, :]).astype(jnp.int32), axis=1),
        E - 1).astype(jnp.int32)

    hw = lax.bitcast_convert_type(h.reshape(N, D // 2, 2), jnp.float32)
    hsw = _sc_gather_rows(hw, row_tok, P)
    h_sorted = lax.bitcast_convert_type(hsw, jnp.bfloat16).reshape(P, D)

    z = _grouped_ffn(h_sorted, tile_expert, g_sorted,
                     W1.astype(jnp.bfloat16), b1,
                     W2.astype(jnp.bfloat16), b2)
    out = _sc_combine(x2, z, pos)
    return out.reshape(B, T, D)


# speedup vs baseline: 1.4316x; 1.4316x over previous
import functools

import jax
import jax.numpy as jnp
from jax import lax
from jax.experimental import pallas as pl
from jax.experimental.pallas import tpu as pltpu
from jax.experimental.pallas import tpu_sc as plsc

TM = 256


def _gelu_exact(x):
    return 0.5 * x * (1.0 + lax.erf(x * 0.7071067811865475))


def _router_body(x_ref, g_ref, b_ref, wr_ref, br_ref,
                 h_ref, gates_ref, idx_ref, *, n_experts):
    xv = x_ref[...]
    mu = jnp.mean(xv, axis=-1, keepdims=True)
    xc = xv - mu
    var = jnp.mean(xc * xc, axis=-1, keepdims=True)
    h = xc * lax.rsqrt(var + 1e-5) * g_ref[0, :] + b_ref[0, :]
    h_ref[...] = h
    logits = jnp.dot(h, wr_ref[...],
                     preferred_element_type=jnp.float32) + br_ref[0, :]
    iota = lax.broadcasted_iota(jnp.int32, logits.shape, 1)
    v1 = jnp.max(logits, axis=-1, keepdims=True)
    i1 = jnp.min(jnp.where(logits >= v1, iota, n_experts),
                 axis=-1, keepdims=True)
    l2 = jnp.where(iota == i1, jnp.float32(-1e30), logits)
    v2 = jnp.max(l2, axis=-1, keepdims=True)
    i2 = jnp.min(jnp.where(l2 >= v2, iota, n_experts),
                 axis=-1, keepdims=True)
    g1 = 1.0 / (1.0 + jnp.exp(v2 - v1))
    col = lax.broadcasted_iota(jnp.int32, (xv.shape[0], 2), 1)
    gates_ref[...] = jnp.where(col == 0, g1, 1.0 - g1)
    idx_ref[...] = jnp.where(col == 0, i1, i2)


def _router(x2, ln_g, ln_b, Wr, br):
    N, D = x2.shape
    E = Wr.shape[1]
    return pl.pallas_call(
        functools.partial(_router_body, n_experts=E),
        out_shape=[
            jax.ShapeDtypeStruct((N, D), jnp.float32),
            jax.ShapeDtypeStruct((N, 2), jnp.float32),
            jax.ShapeDtypeStruct((N, 2), jnp.int32),
        ],
    )(x2, ln_g.reshape(1, D), ln_b.reshape(1, D), Wr, br.reshape(1, E))


def _sc_gather_rows(hw, tok, P):
    N, DW = hw.shape
    info = plsc.get_sparse_core_info()
    NC, NS = info.num_cores, info.num_subcores
    NW = NC * NS
    b_per_w = P // NW
    CH = 80
    n_chunks = b_per_w // CH

    def body(h_hbm, tok_hbm, out_hbm, idx_v, buf0, buf1, sem0, sem1):
        wid = lax.axis_index("s") * NC + lax.axis_index("c")
        base = wid * b_per_w
        pltpu.sync_copy(tok_hbm.at[pl.ds(base, b_per_w)], idx_v)
        bufs, sems = (buf0, buf1), (sem0, sem1)
        pend = [None, None]

        def start(i):
            b = i % 2
            pend[b] = pltpu.async_copy(
                h_hbm.at[idx_v.at[pl.ds(i * CH, CH)]], bufs[b], sems[b])

        start(0)
        for i in range(n_chunks):
            if i + 1 < n_chunks:
                start(i + 1)
            b = i % 2
            pend[b].wait()
            pltpu.sync_copy(bufs[b], out_hbm.at[pl.ds(base + i * CH, CH)])

    return pl.kernel(
        body,
        out_type=jax.ShapeDtypeStruct((P, DW), jnp.float32),
        mesh=plsc.VectorSubcoreMesh(core_axis_name="c", subcore_axis_name="s"),
        scratch_types=[
            pltpu.VMEM((b_per_w,), jnp.int32),
            pltpu.VMEM((CH, DW), jnp.float32),
            pltpu.VMEM((CH, DW), jnp.float32),
            pltpu.SemaphoreType.DMA,
            pltpu.SemaphoreType.DMA,
        ],
    )(hw, tok)


def _ffn_body(te_ref, h_ref, w1_ref, b1_ref, w2_ref, b2_ref, g_ref, z_ref):
    hid = jnp.dot(h_ref[...].astype(jnp.bfloat16), w1_ref[0],
                  preferred_element_type=jnp.float32)
    hid = _gelu_exact(hid + b1_ref[0, 0, :])
    y = jnp.dot(hid.astype(jnp.bfloat16), w2_ref[0],
                preferred_element_type=jnp.float32)
    z_ref[...] = (y + b2_ref[0, 0, :]) * g_ref[...]


def _grouped_ffn(h_sorted, tile_expert, g_sorted, W1b, b1, W2b, b2):
    P, D = h_sorted.shape
    E, _, DFF = W1b.shape
    NT = P // TM
    grid_spec = pltpu.PrefetchScalarGridSpec(
        num_scalar_prefetch=1,
        grid=(NT,),
        in_specs=[
            pl.BlockSpec((TM, D), lambda m, te: (m, 0)),
            pl.BlockSpec((1, D, DFF), lambda m, te: (te[m], 0, 0)),
            pl.BlockSpec((1, 1, DFF), lambda m, te: (te[m], 0, 0)),
            pl.BlockSpec((1, DFF, D), lambda m, te: (te[m], 0, 0)),
            pl.BlockSpec((1, 1, D), lambda m, te: (te[m], 0, 0)),
            pl.BlockSpec((TM, 1), lambda m, te: (m, 0)),
        ],
        out_specs=pl.BlockSpec((TM, D), lambda m, te: (m, 0)),
    )
    return pl.pallas_call(
        _ffn_body,
        grid_spec=grid_spec,
        out_shape=jax.ShapeDtypeStruct((P, D), jnp.float32),
        compiler_params=pltpu.CompilerParams(
            dimension_semantics=("arbitrary",),
            vmem_limit_bytes=100 * 1024 * 1024,
        ),
    )(tile_expert, h_sorted, W1b, b1.reshape(E, 1, DFF), W2b,
      b2.reshape(E, 1, D), g_sorted)


def _sc_combine(x2, z, pos_flat):
    N, D = x2.shape
    info = plsc.get_sparse_core_info()
    NC, NS, L = info.num_cores, info.num_subcores, info.num_lanes
    NW = NC * NS
    b_per_w = N // NW
    CH = 16
    n_chunks = b_per_w // CH
    n_lane_blocks = D // L

    def body(x_hbm, z_hbm, pos_hbm, out_hbm,
             ip_v, xb0, xb1, zb0, zb1, sx0, sx1, sz0, sz1):
        wid = lax.axis_index("s") * NC + lax.axis_index("c")
        base = wid * b_per_w
        pltpu.sync_copy(pos_hbm.at[pl.ds(2 * base, 2 * b_per_w)], ip_v)
        xbufs, zbufs = (xb0, xb1), (zb0, zb1)
        xsems, zsems = (sx0, sx1), (sz0, sz1)
        xpend = [None, None]
        zpend = [None, None]

        def start(i):
            b = i % 2
            xpend[b] = pltpu.async_copy(
                x_hbm.at[pl.ds(base + i * CH, CH)], xbufs[b], xsems[b])
            zpend[b] = pltpu.async_copy(
                z_hbm.at[ip_v.at[pl.ds(2 * i * CH, 2 * CH)]],
                zbufs[b], zsems[b])

        start(0)
        for i in range(n_chunks):
            if i + 1 < n_chunks:
                start(i + 1)
            b = i % 2
            xpend[b].wait()
            zpend[b].wait()
            xb, zb = xbufs[b], zbufs[b]

            def row_add(r, carry):
                for c in range(n_lane_blocks):
                    s = pl.ds(c * L, L)
                    xb[r, s] = xb[r, s] + zb[2 * r, s] + zb[2 * r + 1, s]
                return carry

            lax.fori_loop(0, CH, row_add, 0)
            pltpu.sync_copy(xb, out_hbm.at[pl.ds(base + i * CH, CH)])

    return pl.kernel(
        body,
        out_type=jax.ShapeDtypeStruct((N, D), jnp.float32),
        mesh=plsc.VectorSubcoreMesh(core_axis_name="c", subcore_axis_name="s"),
        scratch_types=[
            pltpu.VMEM((2 * b_per_w,), jnp.int32),
            pltpu.VMEM((CH, D), jnp.float32),
            pltpu.VMEM((CH, D), jnp.float32),
            pltpu.VMEM((2 * CH, D), jnp.float32),
            pltpu.VMEM((2 * CH, D), jnp.float32),
            pltpu.SemaphoreType.DMA,
            pltpu.SemaphoreType.DMA,
            pltpu.SemaphoreType.DMA,
            pltpu.SemaphoreType.DMA,
        ],
    )(x2, z, pos_flat)


def kernel(x, ln_g, ln_b, Wr, br, W1, b1, W2, b2):
    B, T, D = x.shape
    E = Wr.shape[1]
    DFF = W1.shape[2]
    N = B * T
    K = 2
    NK = N * K
    P = NK + E * TM

    x2 = x.reshape(N, D)
    h, gates, idx = _router(x2, ln_g, ln_b, Wr, br)

    idx_flat = idx.reshape(NK)
    onehot = (idx_flat[:, None] == jnp.arange(E, dtype=jnp.int32)
              ).astype(jnp.int32)
    counts = jnp.sum(onehot, axis=0)
    ranks = jnp.sum(onehot * (jnp.cumsum(onehot, axis=0) - onehot),
                    axis=1)
    padded = ((counts + TM - 1) // TM) * TM
    off = jnp.concatenate([jnp.zeros((1,), jnp.int32),
                           jnp.cumsum(padded)[:-1].astype(jnp.int32)])
    pos = off[idx_flat] + ranks
    row_tok = jnp.zeros((P,), jnp.int32).at[pos].set(
        jnp.arange(NK, dtype=jnp.int32) // K)
    g_sorted = jnp.zeros((P,), jnp.float32).at[pos].set(
        gates.reshape(NK)).reshape(P, 1)
    bnd = jnp.cumsum(padded // TM)
    t_iota = jnp.arange(P // TM, dtype=jnp.int32)
    tile_expert = jnp.minimum(
        jnp.sum((t_iota[:, None] >= bnd[None, :]).astype(jnp.int32), axis=1),
        E - 1).astype(jnp.int32)

    h_sorted = _sc_gather_rows(h, row_tok, P)
    z = _grouped_ffn(h_sorted, tile_expert, g_sorted,
                     W1.astype(jnp.bfloat16), b1,
                     W2.astype(jnp.bfloat16), b2)
    out = _sc_combine(x2, z, pos)
    return out.reshape(B, T, D)


# speedup vs baseline: 2.3463x; 1.6390x over previous
import functools

import jax
import jax.numpy as jnp
from jax import lax
from jax.experimental import pallas as pl
from jax.experimental.pallas import tpu as pltpu
from jax.experimental.pallas import tpu_sc as plsc

TM = 256


def _gelu_exact(x):
    return 0.5 * x * (1.0 + lax.erf(x * 0.7071067811865475))


def _router_body(x_ref, g_ref, b_ref, wr_ref, br_ref,
                 h_ref, gates_ref, idx_ref, *, n_experts):
    xv = x_ref[...]
    mu = jnp.mean(xv, axis=-1, keepdims=True)
    xc = xv - mu
    var = jnp.mean(xc * xc, axis=-1, keepdims=True)
    h = xc * lax.rsqrt(var + 1e-5) * g_ref[0, :] + b_ref[0, :]
    h_ref[...] = h
    logits = jnp.dot(h, wr_ref[...],
                     preferred_element_type=jnp.float32) + br_ref[0, :]
    iota = lax.broadcasted_iota(jnp.int32, logits.shape, 1)
    v1 = jnp.max(logits, axis=-1, keepdims=True)
    i1 = jnp.min(jnp.where(logits >= v1, iota, n_experts),
                 axis=-1, keepdims=True)
    l2 = jnp.where(iota == i1, jnp.float32(-1e30), logits)
    v2 = jnp.max(l2, axis=-1, keepdims=True)
    i2 = jnp.min(jnp.where(l2 >= v2, iota, n_experts),
                 axis=-1, keepdims=True)
    g1 = 1.0 / (1.0 + jnp.exp(v2 - v1))
    col = lax.broadcasted_iota(jnp.int32, (xv.shape[0], 2), 1)
    gates_ref[...] = jnp.where(col == 0, g1, 1.0 - g1)
    idx_ref[...] = jnp.where(col == 0, i1, i2)


def _router(x2, ln_g, ln_b, Wr, br):
    N, D = x2.shape
    E = Wr.shape[1]
    return pl.pallas_call(
        functools.partial(_router_body, n_experts=E),
        out_shape=[
            jax.ShapeDtypeStruct((N, D), jnp.float32),
            jax.ShapeDtypeStruct((N, 2), jnp.float32),
            jax.ShapeDtypeStruct((N, 2), jnp.int32),
        ],
    )(x2, ln_g.reshape(1, D), ln_b.reshape(1, D), Wr, br.reshape(1, E))


def _sc_scatter_rows(h, pos0, pos1, P):
    N, D = h.shape
    info = plsc.get_sparse_core_info()
    NC, NS = info.num_cores, info.num_subcores
    NW = NC * NS
    b_per_w = N // NW
    CH = 64
    n_chunks = b_per_w // CH

    def body(h_hbm, p0_hbm, p1_hbm, out_hbm,
             hb0, hb1, i0a, i0b, i1a, i1b,
             hs0, hs1, ss0a, ss0b, ss1a, ss1b):
        wid = lax.axis_index("s") * NC + lax.axis_index("c")
        base = wid * b_per_w
        hbufs, hsems = (hb0, hb1), (hs0, hs1)
        i0r, i1r = (i0a, i0b), (i1a, i1b)
        s0sems, s1sems = (ss0a, ss0b), (ss1a, ss1b)
        hpend = [None, None]
        spend = [None, None]

        def start(i):
            b = i % 2
            if spend[b] is not None:
                spend[b][0].wait()
                spend[b][1].wait()
                spend[b] = None
            hpend[b] = pltpu.async_copy(
                h_hbm.at[pl.ds(base + i * CH, CH)], hbufs[b], hsems[b])

        start(0)
        for i in range(n_chunks):
            b = i % 2
            if i + 1 < n_chunks:
                start(i + 1)
            pltpu.sync_copy(p0_hbm.at[pl.ds(base + i * CH, CH)], i0r[b])
            pltpu.sync_copy(p1_hbm.at[pl.ds(base + i * CH, CH)], i1r[b])
            hpend[b].wait()
            spend[b] = (
                pltpu.async_copy(hbufs[b], out_hbm.at[i0r[b]], s0sems[b]),
                pltpu.async_copy(hbufs[b], out_hbm.at[i1r[b]], s1sems[b]),
            )
        for b in range(2):
            if spend[b] is not None:
                spend[b][0].wait()
                spend[b][1].wait()

    return pl.kernel(
        body,
        out_type=jax.ShapeDtypeStruct((P, D), jnp.float32),
        mesh=plsc.VectorSubcoreMesh(core_axis_name="c", subcore_axis_name="s"),
        scratch_types=[
            pltpu.VMEM((CH, D), jnp.float32),
            pltpu.VMEM((CH, D), jnp.float32),
            pltpu.VMEM((CH,), jnp.int32),
            pltpu.VMEM((CH,), jnp.int32),
            pltpu.VMEM((CH,), jnp.int32),
            pltpu.VMEM((CH,), jnp.int32),
            pltpu.SemaphoreType.DMA,
            pltpu.SemaphoreType.DMA,
            pltpu.SemaphoreType.DMA,
            pltpu.SemaphoreType.DMA,
            pltpu.SemaphoreType.DMA,
            pltpu.SemaphoreType.DMA,
        ],
    )(h, pos0, pos1)


def _ffn_body(te_ref, h_ref, w1_ref, b1_ref, w2_ref, b2_ref, z_ref):
    hid = jnp.dot(h_ref[...].astype(jnp.bfloat16),
                  w1_ref[0].astype(jnp.bfloat16),
                  preferred_element_type=jnp.float32)
    hid = _gelu_exact(hid + b1_ref[0, 0, :])
    y = jnp.dot(hid.astype(jnp.bfloat16), w2_ref[0].astype(jnp.bfloat16),
                preferred_element_type=jnp.float32)
    z_ref[...] = y + b2_ref[0, 0, :]


def _grouped_ffn(h_sorted, tile_expert, W1, b1, W2, b2):
    P, D = h_sorted.shape
    E, _, DFF = W1.shape
    NT = P // TM
    grid_spec = pltpu.PrefetchScalarGridSpec(
        num_scalar_prefetch=1,
        grid=(NT,),
        in_specs=[
            pl.BlockSpec((TM, D), lambda m, te: (m, 0)),
            pl.BlockSpec((1, D, DFF), lambda m, te: (te[m], 0, 0)),
            pl.BlockSpec((1, 1, DFF), lambda m, te: (te[m], 0, 0)),
            pl.BlockSpec((1, DFF, D), lambda m, te: (te[m], 0, 0)),
            pl.BlockSpec((1, 1, D), lambda m, te: (te[m], 0, 0)),
        ],
        out_specs=pl.BlockSpec((TM, D), lambda m, te: (m, 0)),
    )
    return pl.pallas_call(
        _ffn_body,
        grid_spec=grid_spec,
        out_shape=jax.ShapeDtypeStruct((P, D), jnp.float32),
        compiler_params=pltpu.CompilerParams(
            dimension_semantics=("arbitrary",),
            vmem_limit_bytes=100 * 1024 * 1024,
        ),
    )(tile_expert, h_sorted, W1, b1.reshape(E, 1, DFF), W2,
      b2.reshape(E, 1, D))


def _sc_combine(x2, z, pos_flat, g_wide):
    N, D = x2.shape
    info = plsc.get_sparse_core_info()
    NC, NS, L = info.num_cores, info.num_subcores, info.num_lanes
    NW = NC * NS
    b_per_w = N // NW
    CH = 16
    n_chunks = b_per_w // CH
    n_lane_blocks = D // L

    def body(x_hbm, z_hbm, pos_hbm, g_hbm, out_hbm,
             ip_v, xb0, xb1, zb0, zb1, gb0, gb1,
             sx0, sx1, sz0, sz1, sg0, sg1):
        wid = lax.axis_index("s") * NC + lax.axis_index("c")
        base = wid * b_per_w
        pltpu.sync_copy(pos_hbm.at[pl.ds(2 * base, 2 * b_per_w)], ip_v)
        xbufs, zbufs, gbufs = (xb0, xb1), (zb0, zb1), (gb0, gb1)
        xsems, zsems, gsems = (sx0, sx1), (sz0, sz1), (sg0, sg1)
        xpend = [None, None]
        zpend = [None, None]
        gpend = [None, None]

        def start(i):
            b = i % 2
            xpend[b] = pltpu.async_copy(
                x_hbm.at[pl.ds(base + i * CH, CH)], xbufs[b], xsems[b])
            zpend[b] = pltpu.async_copy(
                z_hbm.at[ip_v.at[pl.ds(2 * i * CH, 2 * CH)]],
                zbufs[b], zsems[b])
            gpend[b] = pltpu.async_copy(
                g_hbm.at[pl.ds(2 * (base + i * CH), 2 * CH)],
                gbufs[b], gsems[b])

        start(0)
        for i in range(n_chunks):
            if i + 1 < n_chunks:
                start(i + 1)
            b = i % 2
            xpend[b].wait()
            zpend[b].wait()
            gpend[b].wait()
            xb, zb, gb = xbufs[b], zbufs[b], gbufs[b]

            def row_add(r, carry):
                g0 = gb[2 * r, :]
                g1 = gb[2 * r + 1, :]
                for c in range(n_lane_blocks):
                    s = pl.ds(c * L, L)
                    xb[r, s] = xb[r, s] + g0 * zb[2 * r, s] + g1 * zb[2 * r + 1, s]
                return carry

            lax.fori_loop(0, CH, row_add, 0)
            pltpu.sync_copy(xb, out_hbm.at[pl.ds(base + i * CH, CH)])

    return pl.kernel(
        body,
        out_type=jax.ShapeDtypeStruct((N, D), jnp.float32),
        mesh=plsc.VectorSubcoreMesh(core_axis_name="c", subcore_axis_name="s"),
        scratch_types=[
            pltpu.VMEM((2 * b_per_w,), jnp.int32),
            pltpu.VMEM((CH, D), jnp.float32),
            pltpu.VMEM((CH, D), jnp.float32),
            pltpu.VMEM((2 * CH, D), jnp.float32),
            pltpu.VMEM((2 * CH, D), jnp.float32),
            pltpu.VMEM((2 * CH, L), jnp.float32),
            pltpu.VMEM((2 * CH, L), jnp.float32),
            pltpu.SemaphoreType.DMA,
            pltpu.SemaphoreType.DMA,
            pltpu.SemaphoreType.DMA,
            pltpu.SemaphoreType.DMA,
            pltpu.SemaphoreType.DMA,
            pltpu.SemaphoreType.DMA,
        ],
    )(x2, z, pos_flat, g_wide)


def kernel(x, ln_g, ln_b, Wr, br, W1, b1, W2, b2):
    B, T, D = x.shape
    E = Wr.shape[1]
    DFF = W1.shape[2]
    N = B * T
    K = 2
    NK = N * K
    P = NK + E * TM

    x2 = x.reshape(N, D)
    h, gates, idx = _router(x2, ln_g, ln_b, Wr, br)

    idx_flat = idx.reshape(NK)
    onehot = (idx_flat[:, None] == jnp.arange(E, dtype=jnp.int32)
              ).astype(jnp.float32)
    RB = 128
    NB = NK // RB
    oh3 = onehot.reshape(NB, RB, E)
    tri = (jnp.arange(RB)[:, None] <= jnp.arange(RB)[None, :]
           ).astype(jnp.float32)
    csum = jnp.einsum('bre,rs->bse', oh3, tri)
    blk_tot = csum[:, RB - 1, :]
    trib = (jnp.arange(NB)[:, None] < jnp.arange(NB)[None, :]
            ).astype(jnp.float32)
    blk_off = jnp.einsum('be,bc->ce', blk_tot, trib)
    ranks = jnp.sum(oh3 * (blk_off[:, None, :] + csum - oh3),
                    axis=2).reshape(NK).astype(jnp.int32)
    counts = jnp.sum(blk_tot, axis=0).astype(jnp.int32)
    padded = ((counts + TM - 1) // TM) * TM
    off = jnp.concatenate([jnp.zeros((1,), jnp.int32),
                           jnp.cumsum(padded)[:-1].astype(jnp.int32)])
    pos = off[idx_flat] + ranks
    bnd = jnp.cumsum(padded // TM)
    t_iota = jnp.arange(P // TM, dtype=jnp.int32)
    tile_expert = jnp.minimum(
        jnp.sum((t_iota[:, None] >= bnd[None, :]).astype(jnp.int32), axis=1),
        E - 1).astype(jnp.int32)
    pos2 = pos.reshape(N, K)
    g_wide = jnp.broadcast_to(gates.reshape(NK, 1), (NK, 16))

    h_sorted = _sc_scatter_rows(h, pos2[:, 0], pos2[:, 1], P)
    z = _grouped_ffn(h_sorted, tile_expert, W1, b1, W2, b2)
    out = _sc_combine(x2, z, pos, g_wide)
    return out.reshape(B, T, D)


# speedup vs baseline: 2.4630x; 1.0497x over previous
import functools

import jax
import jax.numpy as jnp
from jax import lax
from jax.experimental import pallas as pl
from jax.experimental.pallas import tpu as pltpu
from jax.experimental.pallas import tpu_sc as plsc

TM = 256


def _gelu_exact(x):
    return 0.5 * x * (1.0 + lax.erf(x * 0.7071067811865475))


_HI_MASK = -65536


def _pack_bf16_pair(a, b):
    au = lax.bitcast_convert_type(
        a.astype(jnp.bfloat16).astype(jnp.float32), jnp.int32)
    bu = lax.bitcast_convert_type(
        b.astype(jnp.bfloat16).astype(jnp.float32), jnp.int32)
    return lax.shift_right_logical(au, 16) | (bu & _HI_MASK)


def _router_body(x_ref, g_ref, b_ref, wr_ref, br_ref,
                 h_ref, gates_ref, idx_ref, *, n_experts):
    xv = x_ref[...]
    mu = jnp.mean(xv, axis=-1, keepdims=True)
    xc = xv - mu
    var = jnp.mean(xc * xc, axis=-1, keepdims=True)
    h = xc * lax.rsqrt(var + 1e-5) * g_ref[0, :] + b_ref[0, :]
    dh = h.shape[1] // 2
    h_ref[...] = _pack_bf16_pair(h[:, :dh], h[:, dh:])
    logits = jnp.dot(h, wr_ref[...],
                     preferred_element_type=jnp.float32) + br_ref[0, :]
    iota = lax.broadcasted_iota(jnp.int32, logits.shape, 1)
    v1 = jnp.max(logits, axis=-1, keepdims=True)
    i1 = jnp.min(jnp.where(logits >= v1, iota, n_experts),
                 axis=-1, keepdims=True)
    l2 = jnp.where(iota == i1, jnp.float32(-1e30), logits)
    v2 = jnp.max(l2, axis=-1, keepdims=True)
    i2 = jnp.min(jnp.where(l2 >= v2, iota, n_experts),
                 axis=-1, keepdims=True)
    g1 = 1.0 / (1.0 + jnp.exp(v2 - v1))
    col = lax.broadcasted_iota(jnp.int32, (xv.shape[0], 2), 1)
    gates_ref[...] = jnp.where(col == 0, g1, 1.0 - g1)
    idx_ref[...] = jnp.where(col == 0, i1, i2)


def _router(x2, ln_g, ln_b, Wr, br):
    N, D = x2.shape
    E = Wr.shape[1]
    return pl.pallas_call(
        functools.partial(_router_body, n_experts=E),
        out_shape=[
            jax.ShapeDtypeStruct((N, D // 2), jnp.int32),
            jax.ShapeDtypeStruct((N, 2), jnp.float32),
            jax.ShapeDtypeStruct((N, 2), jnp.int32),
        ],
    )(x2, ln_g.reshape(1, D), ln_b.reshape(1, D), Wr, br.reshape(1, E))


def _sc_scatter_rows(h, pos0, pos1, P):
    N, D = h.shape
    info = plsc.get_sparse_core_info()
    NC, NS = info.num_cores, info.num_subcores
    NW = NC * NS
    b_per_w = N // NW
    CH = 64
    n_chunks = b_per_w // CH

    def body(h_hbm, p0_hbm, p1_hbm, out_hbm,
             hb0, hb1, i0a, i0b, i1a, i1b,
             hs0, hs1, ss0a, ss0b, ss1a, ss1b):
        wid = lax.axis_index("s") * NC + lax.axis_index("c")
        base = wid * b_per_w
        hbufs, hsems = (hb0, hb1), (hs0, hs1)
        i0r, i1r = (i0a, i0b), (i1a, i1b)
        s0sems, s1sems = (ss0a, ss0b), (ss1a, ss1b)
        hpend = [None, None]
        spend = [None, None]

        def start(i):
            b = i % 2
            if spend[b] is not None:
                spend[b][0].wait()
                spend[b][1].wait()
                spend[b] = None
            hpend[b] = pltpu.async_copy(
                h_hbm.at[pl.ds(base + i * CH, CH)], hbufs[b], hsems[b])

        start(0)
        for i in range(n_chunks):
            b = i % 2
            if i + 1 < n_chunks:
                start(i + 1)
            pltpu.sync_copy(p0_hbm.at[pl.ds(base + i * CH, CH)], i0r[b])
            pltpu.sync_copy(p1_hbm.at[pl.ds(base + i * CH, CH)], i1r[b])
            hpend[b].wait()
            spend[b] = (
                pltpu.async_copy(hbufs[b], out_hbm.at[i0r[b]], s0sems[b]),
                pltpu.async_copy(hbufs[b], out_hbm.at[i1r[b]], s1sems[b]),
            )
        for b in range(2):
            if spend[b] is not None:
                spend[b][0].wait()
                spend[b][1].wait()

    return pl.kernel(
        body,
        out_type=jax.ShapeDtypeStruct((P, D), jnp.int32),
        mesh=plsc.VectorSubcoreMesh(core_axis_name="c", subcore_axis_name="s"),
        scratch_types=[
            pltpu.VMEM((CH, D), jnp.int32),
            pltpu.VMEM((CH, D), jnp.int32),
            pltpu.VMEM((CH,), jnp.int32),
            pltpu.VMEM((CH,), jnp.int32),
            pltpu.VMEM((CH,), jnp.int32),
            pltpu.VMEM((CH,), jnp.int32),
            pltpu.SemaphoreType.DMA,
            pltpu.SemaphoreType.DMA,
            pltpu.SemaphoreType.DMA,
            pltpu.SemaphoreType.DMA,
            pltpu.SemaphoreType.DMA,
            pltpu.SemaphoreType.DMA,
        ],
    )(h, pos0, pos1)


def _ffn_body(te_ref, h_ref, w1_ref, b1_ref, w2_ref, b2_ref, z_ref):
    hu = h_ref[...]
    h_lo = lax.bitcast_convert_type(lax.shift_left(hu, 16), jnp.float32)
    h_hi = lax.bitcast_convert_type(hu & _HI_MASK, jnp.float32)
    h = jnp.concatenate([h_lo, h_hi], axis=1).astype(jnp.bfloat16)
    hid = jnp.dot(h, w1_ref[0].astype(jnp.bfloat16),
                  preferred_element_type=jnp.float32)
    hid = _gelu_exact(hid + b1_ref[0, 0, :])
    y = jnp.dot(hid.astype(jnp.bfloat16), w2_ref[0].astype(jnp.bfloat16),
                preferred_element_type=jnp.float32)
    z_ref[...] = y + b2_ref[0, 0, :]


def _grouped_ffn(h_sorted, tile_expert, W1, b1, W2, b2):
    P, DW = h_sorted.shape
    E, D, DFF = W1.shape
    NT = P // TM
    grid_spec = pltpu.PrefetchScalarGridSpec(
        num_scalar_prefetch=1,
        grid=(NT,),
        in_specs=[
            pl.BlockSpec((TM, DW), lambda m, te: (m, 0)),
            pl.BlockSpec((1, D, DFF), lambda m, te: (te[m], 0, 0)),
            pl.BlockSpec((1, 1, DFF), lambda m, te: (te[m], 0, 0)),
            pl.BlockSpec((1, DFF, D), lambda m, te: (te[m], 0, 0)),
            pl.BlockSpec((1, 1, D), lambda m, te: (te[m], 0, 0)),
        ],
        out_specs=pl.BlockSpec((TM, D), lambda m, te: (m, 0)),
    )
    return pl.pallas_call(
        _ffn_body,
        grid_spec=grid_spec,
        out_shape=jax.ShapeDtypeStruct((P, D), jnp.float32),
        compiler_params=pltpu.CompilerParams(
            dimension_semantics=("arbitrary",),
            vmem_limit_bytes=100 * 1024 * 1024,
        ),
    )(tile_expert, h_sorted, W1, b1.reshape(E, 1, DFF), W2,
      b2.reshape(E, 1, D))


def _sc_combine(x2, z, pos_flat, g_wide):
    N, D = x2.shape
    info = plsc.get_sparse_core_info()
    NC, NS, L = info.num_cores, info.num_subcores, info.num_lanes
    NW = NC * NS
    b_per_w = N // NW
    CH = 16
    n_chunks = b_per_w // CH
    n_lane_blocks = D // L

    def body(x_hbm, z_hbm, pos_hbm, g_hbm, out_hbm,
             ip_v, xb0, xb1, zb0, zb1, gb0, gb1,
             sx0, sx1, sz0, sz1, sg0, sg1):
        wid = lax.axis_index("s") * NC + lax.axis_index("c")
        base = wid * b_per_w
        pltpu.sync_copy(pos_hbm.at[pl.ds(2 * base, 2 * b_per_w)], ip_v)
        xbufs, zbufs, gbufs = (xb0, xb1), (zb0, zb1), (gb0, gb1)
        xsems, zsems, gsems = (sx0, sx1), (sz0, sz1), (sg0, sg1)
        xpend = [None, None]
        zpend = [None, None]
        gpend = [None, None]

        def start(i):
            b = i % 2
            xpend[b] = pltpu.async_copy(
                x_hbm.at[pl.ds(base + i * CH, CH)], xbufs[b], xsems[b])
            zpend[b] = pltpu.async_copy(
                z_hbm.at[ip_v.at[pl.ds(2 * i * CH, 2 * CH)]],
                zbufs[b], zsems[b])
            gpend[b] = pltpu.async_copy(
                g_hbm.at[pl.ds(2 * (base + i * CH), 2 * CH)],
                gbufs[b], gsems[b])

        start(0)
        for i in range(n_chunks):
            if i + 1 < n_chunks:
                start(i + 1)
            b = i % 2
            xpend[b].wait()
            zpend[b].wait()
            gpend[b].wait()
            xb, zb, gb = xbufs[b], zbufs[b], gbufs[b]

            def row_add(r, carry):
                g0 = gb[2 * r, :]
                g1 = gb[2 * r + 1, :]
                for c in range(n_lane_blocks):
                    s = pl.ds(c * L, L)
                    xb[r, s] = xb[r, s] + g0 * zb[2 * r, s] + g1 * zb[2 * r + 1, s]
                return carry

            lax.fori_loop(0, CH, row_add, 0)
            pltpu.sync_copy(xb, out_hbm.at[pl.ds(base + i * CH, CH)])

    return pl.kernel(
        body,
        out_type=jax.ShapeDtypeStruct((N, D), jnp.float32),
        mesh=plsc.VectorSubcoreMesh(core_axis_name="c", subcore_axis_name="s"),
        scratch_types=[
            pltpu.VMEM((2 * b_per_w,), jnp.int32),
            pltpu.VMEM((CH, D), jnp.float32),
            pltpu.VMEM((CH, D), jnp.float32),
            pltpu.VMEM((2 * CH, D), jnp.float32),
            pltpu.VMEM((2 * CH, D), jnp.float32),
            pltpu.VMEM((2 * CH, L), jnp.float32),
            pltpu.VMEM((2 * CH, L), jnp.float32),
            pltpu.SemaphoreType.DMA,
            pltpu.SemaphoreType.DMA,
            pltpu.SemaphoreType.DMA,
            pltpu.SemaphoreType.DMA,
            pltpu.SemaphoreType.DMA,
            pltpu.SemaphoreType.DMA,
        ],
    )(x2, z, pos_flat, g_wide)


def kernel(x, ln_g, ln_b, Wr, br, W1, b1, W2, b2):
    B, T, D = x.shape
    E = Wr.shape[1]
    DFF = W1.shape[2]
    N = B * T
    K = 2
    NK = N * K
    P = NK + E * TM

    x2 = x.reshape(N, D)
    h, gates, idx = _router(x2, ln_g, ln_b, Wr, br)

    idx_flat = idx.reshape(NK)
    onehot = (idx_flat[:, None] == jnp.arange(E, dtype=jnp.int32)
              ).astype(jnp.float32)
    RB = 128
    NB = NK // RB
    oh3 = onehot.reshape(NB, RB, E)
    tri = (jnp.arange(RB)[:, None] <= jnp.arange(RB)[None, :]
           ).astype(jnp.float32)
    csum = jnp.einsum('bre,rs->bse', oh3, tri)
    blk_tot = csum[:, RB - 1, :]
    trib = (jnp.arange(NB)[:, None] < jnp.arange(NB)[None, :]
            ).astype(jnp.float32)
    blk_off = jnp.einsum('be,bc->ce', blk_tot, trib)
    ranks = jnp.sum(oh3 * (blk_off[:, None, :] + csum - oh3),
                    axis=2).reshape(NK).astype(jnp.int32)
    counts = jnp.sum(blk_tot, axis=0).astype(jnp.int32)
    padded = ((counts + TM - 1) // TM) * TM
    off = jnp.concatenate([jnp.zeros((1,), jnp.int32),
                           jnp.cumsum(padded)[:-1].astype(jnp.int32)])
    pos = off[idx_flat] + ranks
    bnd = jnp.cumsum(padded // TM)
    t_iota = jnp.arange(P // TM, dtype=jnp.int32)
    tile_expert = jnp.minimum(
        jnp.sum((t_iota[:, None] >= bnd[None, :]).astype(jnp.int32), axis=1),
        E - 1).astype(jnp.int32)
    pos2 = pos.reshape(N, K)
    g_wide = jnp.broadcast_to(gates.reshape(NK, 1), (NK, 16))

    h_sorted = _sc_scatter_rows(h, pos2[:, 0], pos2[:, 1], P)
    z = _grouped_ffn(h_sorted, tile_expert, W1, b1, W2, b2)
    out = _sc_combine(x2, z, pos, g_wide)
    return out.reshape(B, T, D)
